# baseline (speedup 1.0000x reference)
import jax
import jax.numpy as jnp
from jax import lax
from jax.experimental import pallas as pl
from jax.experimental.pallas import tpu as pltpu
from jax.experimental.pallas import tpu_sc as plsc

_N_EMB = 8192
_EMB_DIM = 32
_M = 32 * 32 * 32
_BM = 256
_NB = _M // _BM

_NW = 32
_BPW = _M // _NW
_CHUNK = 128


_W = 4096


_CH = 1024


def _argmin_body(zf_ref, emb_ref, e2_ref, z2_ref, idx_ref, dsum_ref):
    i = pl.program_id(0)
    zf = zf_ref[...]
    z2 = z2_ref[...]
    e2 = e2_ref[...]

    _OFF = 0x30000000
    acc_v = None
    for w in range(_N_EMB // _W):
        wv = None
        for ch in range(_W // _CH):
            lo = w * _W + ch * _CH
            mm = lax.dot_general(
                zf, emb_ref[pl.ds(lo, _CH), :], (((1,), (1,)), ((), ())),
                preferred_element_type=jnp.float32)
            dc = (z2 + e2[:, lo:lo + _CH]) - 2.0 * mm
            cv = jnp.min(dc, axis=1, keepdims=True)
            lane = (lax.broadcasted_iota(jnp.int32, dc.shape, 1)
                    + (lo + _OFF))
            key = lax.bitcast_convert_type(
                jnp.where(dc == cv, lane, _N_EMB + _OFF), jnp.float32)
            ci = jnp.min(key, axis=1, keepdims=True)
            if wv is None:
                wv, wi_f = cv, ci
            else:
                u = cv < wv
                wi_f = jnp.where(u, ci, wi_f)
                wv = jnp.where(u, cv, wv)
        wi = lax.bitcast_convert_type(wi_f, jnp.int32) - _OFF
        if acc_v is None:
            acc_v, acc_i, acc_vx = wv, wi, wv
        else:
            upd = wv < acc_v
            acc_i = jnp.where(upd, wi, acc_i)
            acc_vx = jnp.where(upd, wv, acc_vx)
            acc_v = jnp.where(upd, wv, acc_v)
        acc_v = acc_v.astype(jnp.bfloat16).astype(jnp.float32)
    idx_ref[0, 0, :] = acc_i[:, 0]

    @pl.when(i == 0)
    def _():
        dsum_ref[0, 0] = 0.0

    dsum_ref[0, 0] += jnp.sum(acc_vx)


def _argmin_call(zf, emb, e2, z2):
    return pl.pallas_call(
        _argmin_body,
        grid=(_NB,),
        in_specs=[
            pl.BlockSpec((_BM, _EMB_DIM), lambda i: (i, 0)),
            pl.BlockSpec((_N_EMB, _EMB_DIM), lambda i: (0, 0)),
            pl.BlockSpec((1, _N_EMB), lambda i: (0, 0)),
            pl.BlockSpec((_BM, 1), lambda i: (i, 0)),
        ],
        out_specs=[
            pl.BlockSpec((1, 1, _BM), lambda i: (i, 0, 0)),
            pl.BlockSpec((1, 1), lambda i: (0, 0), memory_space=pltpu.SMEM),
        ],
        out_shape=[
            jax.ShapeDtypeStruct((_NB, 1, _BM), jnp.int32),
            jax.ShapeDtypeStruct((1, 1), jnp.float32),
        ],
    )(zf, emb, e2, z2)


_B = 32
_HW = 32 * 32


def _stgather_body(embT_hbm, idx_hbm, z_hbm, out_hbm, row_v, idx_v, zc_v,
                   out_v):
    c = lax.axis_index("s") * 2 + lax.axis_index("c")
    pltpu.sync_copy(embT_hbm.at[c], row_v)
    pltpu.sync_copy(idx_hbm, idx_v)
    pltpu.sync_copy(z_hbm.at[:, c], zc_v)

    for b in range(_B):
        def body(t, carry):
            sl = pl.ds(t * 16, 16)
            iv = idx_v[b, sl]
            g = plsc.load_gather(row_v, [iv])
            zv = zc_v[b, sl]
            out_v[b, sl] = zv + (g - zv)
            return carry
        lax.fori_loop(0, _HW // 16, body, 0)

    pltpu.sync_copy(out_v, out_hbm.at[:, c])


def _stgather_call(embT, idx2, z3):
    mesh = plsc.VectorSubcoreMesh(core_axis_name="c", subcore_axis_name="s")
    k = pl.kernel(
        _stgather_body,
        mesh=mesh,
        compiler_params=pltpu.CompilerParams(use_tc_tiling_on_sc=False,
                                             needs_layout_passes=False),
        out_type=jax.ShapeDtypeStruct((_B, _EMB_DIM, _HW), jnp.float32),
        scratch_types=[
            pltpu.VMEM((_N_EMB,), jnp.float32),
            pltpu.VMEM((_B, _HW), jnp.int32),
            pltpu.VMEM((_B, _HW), jnp.float32),
            pltpu.VMEM((_B, _HW), jnp.float32),
        ],
    )
    return k(embT, idx2, z3)


def kernel(z, emb):
    zp = jnp.transpose(z, (0, 2, 3, 1))
    zf = zp.reshape(-1, _EMB_DIM)
    e2 = jnp.sum(emb ** 2, axis=1).reshape(1, _N_EMB)
    z2 = jnp.sum(z ** 2, axis=1).reshape(_M, 1)
    idx3, dsum = _argmin_call(zf, emb, e2, z2)
    idx = idx3.reshape(_M)
    embT = emb.T
    zq3 = _stgather_call(embT, idx.reshape(_B, _HW),
                         z.reshape(_B, _EMB_DIM, _HW))
    embedding_loss = dsum[0, 0] * (1.0 / float(_M * _EMB_DIM))
    z_q_out = zq3.reshape(z.shape)
    return (embedding_loss, z_q_out, idx)


# speedup vs baseline: 1.0552x; 1.0552x over previous
import jax
import jax.numpy as jnp
from jax import lax
from jax.experimental import pallas as pl
from jax.experimental.pallas import tpu as pltpu
from jax.experimental.pallas import tpu_sc as plsc

_N_EMB = 8192
_EMB_DIM = 32
_M = 32 * 32 * 32
_BM = 256
_NB = _M // _BM

_W = 4096


def _argmin_body(zf_ref, emb_ref, e2_ref, z2_ref, idx_ref, dsum_ref):
    i = pl.program_id(0)
    zf = zf_ref[...]
    mm = lax.dot_general(zf, emb_ref[...], (((1,), (1,)), ((), ())),
                         preferred_element_type=jnp.float32)
    z2 = z2_ref[...]
    d = (z2 + e2_ref[...]) - 2.0 * mm

    _OFF = 0x30000000
    acc_v = None
    for w in range(_N_EMB // _W):
        dw = d[:, w * _W:(w + 1) * _W]
        wv = jnp.min(dw, axis=1, keepdims=True)
        lane = lax.broadcasted_iota(jnp.int32, dw.shape, 1) + (w * _W + _OFF)
        key = lax.bitcast_convert_type(
            jnp.where(dw == wv, lane, _N_EMB + _OFF), jnp.float32)
        wi_f = jnp.min(key, axis=1, keepdims=True)
        wi = lax.bitcast_convert_type(wi_f, jnp.int32) - _OFF
        if acc_v is None:
            acc_v, acc_i, acc_vx = wv, wi, wv
        else:
            upd = wv < acc_v
            acc_i = jnp.where(upd, wi, acc_i)
            acc_vx = jnp.where(upd, wv, acc_vx)
            acc_v = jnp.where(upd, wv, acc_v)
        acc_v = acc_v.astype(jnp.bfloat16).astype(jnp.float32)
    idx_ref[0, 0, :] = acc_i[:, 0]

    @pl.when(i == 0)
    def _():
        dsum_ref[0, 0] = 0.0

    dsum_ref[0, 0] += jnp.sum(acc_vx)


def _argmin_call(zf, emb, e2, z2):
    return pl.pallas_call(
        _argmin_body,
        grid=(_NB,),
        in_specs=[
            pl.BlockSpec((_BM, _EMB_DIM), lambda i: (i, 0)),
            pl.BlockSpec((_N_EMB, _EMB_DIM), lambda i: (0, 0)),
            pl.BlockSpec((1, _N_EMB), lambda i: (0, 0)),
            pl.BlockSpec((_BM, 1), lambda i: (i, 0)),
        ],
        out_specs=[
            pl.BlockSpec((1, 1, _BM), lambda i: (i, 0, 0)),
            pl.BlockSpec((1, 1), lambda i: (0, 0), memory_space=pltpu.SMEM),
        ],
        out_shape=[
            jax.ShapeDtypeStruct((_NB, 1, _BM), jnp.int32),
            jax.ShapeDtypeStruct((1, 1), jnp.float32),
        ],
    )(zf, emb, e2, z2)


_B = 32
_HW = 32 * 32


def _stgather_body(embT_hbm, idx_hbm, z_hbm, out_hbm, row_v, idx_v, zc_v,
                   out_v):
    c = lax.axis_index("s") * 2 + lax.axis_index("c")
    pltpu.sync_copy(embT_hbm.at[c], row_v)
    pltpu.sync_copy(idx_hbm, idx_v)
    pltpu.sync_copy(z_hbm.at[:, c], zc_v)

    for b in range(_B):
        def body(t, carry):
            sl = pl.ds(t * 16, 16)
            iv = idx_v[b, sl]
            g = plsc.load_gather(row_v, [iv])
            zv = zc_v[b, sl]
            out_v[b, sl] = zv + (g - zv)
            return carry
        lax.fori_loop(0, _HW // 16, body, 0)

    pltpu.sync_copy(out_v, out_hbm.at[:, c])


def _stgather_call(embT, idx2, z3):
    mesh = plsc.VectorSubcoreMesh(core_axis_name="c", subcore_axis_name="s")
    k = pl.kernel(
        _stgather_body,
        mesh=mesh,
        compiler_params=pltpu.CompilerParams(use_tc_tiling_on_sc=False,
                                             needs_layout_passes=False),
        out_type=jax.ShapeDtypeStruct((_B, _EMB_DIM, _HW), jnp.float32),
        scratch_types=[
            pltpu.VMEM((_N_EMB,), jnp.float32),
            pltpu.VMEM((_B, _HW), jnp.int32),
            pltpu.VMEM((_B, _HW), jnp.float32),
            pltpu.VMEM((_B, _HW), jnp.float32),
        ],
    )
    return k(embT, idx2, z3)


def kernel(z, emb):
    zp = jnp.transpose(z, (0, 2, 3, 1))
    zf = zp.reshape(-1, _EMB_DIM)
    e2 = jnp.sum(emb ** 2, axis=1).reshape(1, _N_EMB)
    z2 = jnp.sum(z ** 2, axis=1).reshape(_M, 1)
    idx3, dsum = _argmin_call(zf, emb, e2, z2)
    idx = idx3.reshape(_M)
    embT = emb.T
    zq3 = _stgather_call(embT, idx.reshape(_B, _HW),
                         z.reshape(_B, _EMB_DIM, _HW))
    embedding_loss = dsum[0, 0] * (1.0 / float(_M * _EMB_DIM))
    z_q_out = zq3.reshape(z.shape)
    return (embedding_loss, z_q_out, idx)
